# baseline (speedup 1.0000x reference)
import numpy as np
import jax
import jax.numpy as jnp
from jax.experimental import pallas as pl
from jax.experimental.pallas import tpu as pltpu

_GRAD_THRESHOLD = 0.5
_MIN_OPACITY = 0.05
_LOG2 = float(np.log(2.0))


def _norm_body(n, m, sc_ref, out_ref):
    i = pl.program_id(0)
    s = jnp.exp(sc_ref[...])
    n2 = jnp.sum(s * s, axis=0, keepdims=True)
    col = i * m + jax.lax.broadcasted_iota(jnp.int32, (1, m), 1)
    norm = jnp.where(col < n, jnp.sqrt(n2), jnp.float32(np.inf))
    out_ref[...] = norm[None]


def _median_body(k1, k2, x_ref, thr_ref):
    x = x_ref[...]
    xi = jax.lax.bitcast_convert_type(x, jnp.int32)

    def cnt_le(t):
        return jnp.sum((xi <= t).astype(jnp.int32))

    def it(_, carry):
        lo, hi = carry
        mid = lo + (hi - lo) // 2
        pred = cnt_le(mid) >= k1
        lo2 = jnp.where(pred, lo, mid)
        hi2 = jnp.where(pred, mid, hi)
        return lo2, hi2

    lo0 = jnp.int32(-1)
    hi0 = jnp.int32(0x7F800000)
    _, a_int = jax.lax.fori_loop(0, 31, it, (lo0, hi0))
    neg_inf = jnp.float32(-np.inf)
    pos_inf = jnp.float32(np.inf)
    a = jnp.max(jnp.where(xi <= a_int, x, neg_inf))
    c_a = cnt_le(a_int)
    b = jnp.where(c_a >= k2, a, jnp.min(jnp.where(xi > a_int, x, pos_inf)))
    thr_ref[0, 0] = (a + b) * 0.5


def _main_body(thr_ref, x_ref, out_ref):
    thr = thr_ref[0, 0]
    x = x_ref[...]
    p = x[0:23]
    pos = x[0:3]
    sc = x[3:6]
    opac = x[10:11]
    ga = x[23:25]
    gcf = x[25:26]

    cnts = jnp.maximum(gcf, 1.0)
    avg = ga / cnts
    gn2 = jnp.sum(avg * avg, axis=0, keepdims=True)
    large = gn2 >= _GRAD_THRESHOLD * _GRAD_THRESHOLD
    asc = jnp.exp(sc)
    snorm = jnp.sqrt(jnp.sum(asc * asc, axis=0, keepdims=True))
    clone = large & (snorm <= thr)
    split = large & (snorm > thr)
    act_op = jax.nn.sigmoid(opac)
    keep = jnp.logical_not((act_op < _MIN_OPACITY) | split)

    zero = jnp.float32(0.0)
    kf = jnp.where(keep, 1.0, zero)
    cf = jnp.where(clone, 1.0, zero)
    sf = jnp.where(split, 1.0, zero)

    out_ref[0] = p * kf
    out_ref[1] = p * cf
    sp_sc = (sc - _LOG2) * sf
    tail = x[6:23] * sf
    for i in range(2):
        sn = x[26 + 3 * i:29 + 3 * i]
        out_ref[2 + i, 0:3] = (pos + sn * asc) * sf
        out_ref[2 + i, 3:6] = sp_sc
        out_ref[2 + i, 6:23] = tail


def _pick_block(npad, cap):
    best = 128
    k = 128
    while k <= cap:
        if npad % k == 0:
            best = k
        k += 128
    return best


def _build(n, interpret=False):
    f32 = jnp.float32
    npad = ((n + 1023) // 1024) * 1024
    n8 = npad // 8
    norms_call = pl.pallas_call(
        lambda sc_ref, out_ref: _norm_body(n, n8, sc_ref, out_ref),
        grid=(8,),
        in_specs=[pl.BlockSpec((3, n8), lambda i: (0, i))],
        out_specs=pl.BlockSpec((1, 1, n8), lambda i: (i, 0, 0)),
        out_shape=jax.ShapeDtypeStruct((8, 1, n8), f32),
        interpret=interpret,
    )

    k1 = n // 2
    k2 = n // 2 + 1
    median_call = pl.pallas_call(
        lambda x_ref, t_ref: _median_body(k1, k2, x_ref, t_ref),
        in_specs=[pl.BlockSpec(memory_space=pltpu.VMEM)],
        out_specs=pl.BlockSpec(memory_space=pltpu.SMEM),
        out_shape=jax.ShapeDtypeStruct((1, 1), f32),
        interpret=interpret,
    )

    b = _pick_block(npad, 25600)
    nb = npad // b
    main_call = pl.pallas_call(
        _main_body,
        grid=(nb,),
        in_specs=[
            pl.BlockSpec(memory_space=pltpu.SMEM),
            pl.BlockSpec((32, b), lambda i: (0, i)),
        ],
        out_specs=pl.BlockSpec((4, 23, b), lambda i: (0, 0, i)),
        out_shape=jax.ShapeDtypeStruct((4, 23, npad), f32),
        interpret=interpret,
    )

    def run(positions, scales, rotations, opacities, sh_dc, sh_rest,
            grad_accum, grad_count, split_noise):
        sct = scales.T
        gcf = grad_count.astype(f32)[None, :]
        packed = jnp.concatenate(
            [positions.T, sct, rotations.T, opacities.T, sh_dc.T,
             sh_rest.T, grad_accum.T, gcf, split_noise[0].T,
             split_noise[1].T], axis=0)
        pad = npad - n
        packed = jnp.pad(packed, ((0, 0), (0, pad)))
        norms = norms_call(packed[3:6])
        thr = median_call(norms.reshape(8, n8))
        out4 = main_call(thr, packed)
        return jnp.transpose(out4[:, :, :n], (0, 2, 1)).reshape(4 * n, 23)

    return run


_CACHE = {}


def kernel(positions, scales, rotations, opacities, sh_dc, sh_rest,
           grad_accum, grad_count, split_noise):
    n = positions.shape[0]
    if n not in _CACHE:
        _CACHE[n] = _build(n)
    return _CACHE[n](positions, scales, rotations, opacities, sh_dc, sh_rest,
                     grad_accum, grad_count, split_noise)


# speedup vs baseline: 2.8326x; 2.8326x over previous
import numpy as np
import jax
import jax.numpy as jnp
from jax.experimental import pallas as pl
from jax.experimental.pallas import tpu as pltpu

_GRAD_THRESHOLD = 0.5
_MIN_OPACITY = 0.05
_LOG2 = float(np.log(2.0))


def _norm_body(n, b, sc_ref, out_ref):
    i = pl.program_id(0)
    s = jnp.exp(jnp.transpose(sc_ref[...]))
    n2 = jnp.sum(s * s, axis=0, keepdims=True)
    col = i * b + jax.lax.broadcasted_iota(jnp.int32, (1, b), 1)
    norm = jnp.where(col < n, jnp.sqrt(n2), jnp.float32(np.inf))
    out_ref[...] = norm[None]


def _median_body(k1, k2, x_ref, thr_ref):
    x = x_ref[...]
    xi = jax.lax.bitcast_convert_type(x, jnp.int32)

    def cnt_le(t):
        return jnp.sum((xi <= t).astype(jnp.int32))

    def it(_, carry):
        lo, hi = carry
        mid = lo + (hi - lo) // 2
        pred = cnt_le(mid) >= k1
        lo2 = jnp.where(pred, lo, mid)
        hi2 = jnp.where(pred, mid, hi)
        return lo2, hi2

    lo0 = jnp.int32(-1)
    hi0 = jnp.int32(0x7F800000)
    _, a_int = jax.lax.fori_loop(0, 31, it, (lo0, hi0))
    neg_inf = jnp.float32(-np.inf)
    pos_inf = jnp.float32(np.inf)
    a = jnp.max(jnp.where(xi <= a_int, x, neg_inf))
    c_a = cnt_le(a_int)
    b = jnp.where(c_a >= k2, a, jnp.min(jnp.where(xi > a_int, x, pos_inf)))
    thr_ref[0, 0] = (a + b) * 0.5


def _main_body(thr_ref, pos_ref, sc_ref, rot_ref, op_ref, dc_ref, rest_ref,
               ga_ref, gc_ref, sn_ref, out_ref):
    thr = thr_ref[0, 0]
    t = jnp.transpose
    pos = t(pos_ref[...])
    sc = t(sc_ref[...])
    ga = t(ga_ref[...])
    gcf = t(gc_ref[...].astype(jnp.float32))
    opac = t(op_ref[...])

    cnts = jnp.maximum(gcf, 1.0)
    avg = ga / cnts
    gn2 = jnp.sum(avg * avg, axis=0, keepdims=True)
    large = gn2 >= _GRAD_THRESHOLD * _GRAD_THRESHOLD
    asc = jnp.exp(sc)
    snorm = jnp.sqrt(jnp.sum(asc * asc, axis=0, keepdims=True))
    clone = large & (snorm <= thr)
    split = large & (snorm > thr)
    act_op = jax.nn.sigmoid(opac)
    keep = jnp.logical_not((act_op < _MIN_OPACITY) | split)

    zero = jnp.float32(0.0)
    kf = jnp.where(keep, 1.0, zero)
    cf = jnp.where(clone, 1.0, zero)
    sf = jnp.where(split, 1.0, zero)

    p = jnp.concatenate(
        [pos, sc, t(rot_ref[...]), opac, t(dc_ref[...]), t(rest_ref[...])],
        axis=0)
    out_ref[0] = t(p * kf)
    out_ref[1] = t(p * cf)
    sp_sc = sc - _LOG2
    tail = p[6:23]
    for i in range(2):
        sn = t(sn_ref[i])
        pi = jnp.concatenate([pos + sn * asc, sp_sc, tail], axis=0)
        out_ref[2 + i] = t(pi * sf)


def _build(n, interpret=False):
    f32 = jnp.float32
    b = 25600 if n >= 25600 else ((n + 7) // 8) * 8
    nb = -(-n // b)
    npad = nb * b

    norms_call = pl.pallas_call(
        lambda sc_ref, out_ref: _norm_body(n, b, sc_ref, out_ref),
        grid=(nb,),
        in_specs=[pl.BlockSpec((b, 3), lambda i: (i, 0))],
        out_specs=pl.BlockSpec((1, 1, b), lambda i: (i, 0, 0)),
        out_shape=jax.ShapeDtypeStruct((nb, 1, b), f32),
        interpret=interpret,
    )

    k1 = n // 2
    k2 = n // 2 + 1
    median_call = pl.pallas_call(
        lambda x_ref, t_ref: _median_body(k1, k2, x_ref, t_ref),
        in_specs=[pl.BlockSpec(memory_space=pltpu.VMEM)],
        out_specs=pl.BlockSpec(memory_space=pltpu.SMEM),
        out_shape=jax.ShapeDtypeStruct((1, 1), f32),
        interpret=interpret,
    )

    bm = min(3200, b)
    nbm = -(-n // bm)
    main_call = pl.pallas_call(
        _main_body,
        grid=(nbm,),
        in_specs=[
            pl.BlockSpec(memory_space=pltpu.SMEM),
            pl.BlockSpec((bm, 3), lambda i: (i, 0)),
            pl.BlockSpec((bm, 3), lambda i: (i, 0)),
            pl.BlockSpec((bm, 4), lambda i: (i, 0)),
            pl.BlockSpec((bm, 1), lambda i: (i, 0)),
            pl.BlockSpec((bm, 3), lambda i: (i, 0)),
            pl.BlockSpec((bm, 9), lambda i: (i, 0)),
            pl.BlockSpec((bm, 2), lambda i: (i, 0)),
            pl.BlockSpec((bm, 1), lambda i: (i, 0)),
            pl.BlockSpec((2, bm, 3), lambda i: (0, i, 0)),
        ],
        out_specs=pl.BlockSpec((4, bm, 23), lambda i: (0, i, 0)),
        out_shape=jax.ShapeDtypeStruct((4, n, 23), f32),
        interpret=interpret,
    )

    def run(positions, scales, rotations, opacities, sh_dc, sh_rest,
            grad_accum, grad_count, split_noise):
        norms = norms_call(scales)
        thr = median_call(norms.reshape(8, npad // 8))
        out4 = main_call(thr, positions, scales, rotations, opacities,
                         sh_dc, sh_rest, grad_accum,
                         grad_count.reshape(n, 1), split_noise)
        return out4.reshape(4 * n, 23)

    return run


_CACHE = {}


def kernel(positions, scales, rotations, opacities, sh_dc, sh_rest,
           grad_accum, grad_count, split_noise):
    n = positions.shape[0]
    if n not in _CACHE:
        _CACHE[n] = _build(n)
    return _CACHE[n](positions, scales, rotations, opacities, sh_dc, sh_rest,
                     grad_accum, grad_count, split_noise)
